# baseline (speedup 1.0000x reference)
import functools

import jax
import jax.numpy as jnp
from jax import lax
from jax.experimental import pallas as pl
from jax.experimental.pallas import tpu as pltpu
from jax.experimental.pallas import tpu_sc as plsc

N = 10000
D = 128
E = 320000
NPAD = 10112
ROWS_PER_TILE = NPAD // 16
K = 128
NW = 32
GB = 2
CHUNKS = 80
EPW = CHUNKS * K
EPAD = NW * EPW
GROUPS = CHUNKS // GB


def _agg_body(h_hbm, src_hbm, dst_hbm, zeros_hbm, out_hbm,
              src_v, dst0, dst1, rows_v,
              sem_i, sem_g, sem_s, acc):
    c = lax.axis_index("c")
    s = lax.axis_index("s")
    wid = s * 2 + c
    dsts = (dst0, dst1)
    r0 = s * ROWS_PER_TILE
    pltpu.sync_copy(zeros_hbm.at[pl.ds(r0, ROWS_PER_TILE)],
                    acc.at[pl.ds(r0, ROWS_PER_TILE)])
    plsc.subcore_barrier()

    base = wid * EPW

    def group(j, carry):
        @pl.when(j > 0)
        def _():
            for b in range(GB):
                pltpu.make_async_copy(zeros_hbm.at[pl.ds(0, K)],
                                      rows_v.at[b], sem_s).wait()

        idx_descs = []
        for b in range(GB):
            off = base + (j * GB + b) * K
            d1 = pltpu.async_copy(src_hbm.at[pl.ds(off, K)],
                                  src_v.at[b], sem_i)
            d2 = pltpu.async_copy(dst_hbm.at[pl.ds(off, K)],
                                  dsts[b], sem_i)
            idx_descs.append((d1, d2))
        g_descs = []
        for b in range(GB):
            idx_descs[b][0].wait()
            idx_descs[b][1].wait()
            g_descs.append(
                pltpu.async_copy(h_hbm.at[src_v.at[b]], rows_v.at[b], sem_g))
        for b in range(GB):
            g_descs[b].wait()
            pltpu.async_copy(rows_v.at[b], acc.at[dsts[b]], sem_s, add=True)
        return carry

    lax.fori_loop(0, GROUPS, group, 0)
    for b in range(GB):
        pltpu.make_async_copy(zeros_hbm.at[pl.ds(0, K)],
                              rows_v.at[b], sem_s).wait()
    plsc.subcore_barrier()
    pltpu.sync_copy(acc.at[pl.ds(r0, ROWS_PER_TILE)],
                    out_hbm.at[c, pl.ds(r0, ROWS_PER_TILE)])


@functools.cache
def _make_agg():
    return pl.kernel(
        _agg_body,
        out_type=jax.ShapeDtypeStruct((2, NPAD, D), jnp.float32),
        mesh=plsc.VectorSubcoreMesh(core_axis_name="c", subcore_axis_name="s"),
        scratch_types=[
            pltpu.VMEM((GB, K), jnp.int32),
            pltpu.VMEM((K,), jnp.int32),
            pltpu.VMEM((K,), jnp.int32),
            pltpu.VMEM((GB, K, D), jnp.float32),
            pltpu.SemaphoreType.DMA,
            pltpu.SemaphoreType.DMA,
            pltpu.SemaphoreType.DMA,
            pltpu.VMEM_SHARED((NPAD, D), jnp.float32),
        ],
    )


def _bn(z, g, b):
    mu = jnp.mean(z, axis=0, keepdims=True)
    var = jnp.mean((z - mu) ** 2, axis=0, keepdims=True)
    return (z - mu) * lax.rsqrt(var + 1e-5) * g + b


def _mlp(z, w1, b1, w2, b2):
    z = jnp.maximum(
        jnp.dot(z, w1, preferred_element_type=jnp.float32) + b1, 0.0)
    return jnp.dot(z, w2, preferred_element_type=jnp.float32) + b2


def _combine_body(h_ref, p_ref, z_ref):
    z_ref[...] = h_ref[...] + p_ref[0, :N, :] + p_ref[1, :N, :]


_combine = pl.pallas_call(
    _combine_body,
    out_shape=jax.ShapeDtypeStruct((N, D), jnp.float32),
)


def _layer_body(z_ref, w1_ref, b1_ref, w2_ref, b2_ref, g_ref, be_ref,
                out_ref, *, act):
    z = _mlp(z_ref[...], w1_ref[...], b1_ref[...], w2_ref[...], b2_ref[...])
    z = jnp.maximum(z, 0.0) if act == "relu" else jnp.tanh(z)
    out_ref[...] = _bn(z, g_ref[...], be_ref[...])


_layer = pl.pallas_call(
    functools.partial(_layer_body, act="relu"),
    out_shape=jax.ShapeDtypeStruct((N, D), jnp.float32),
)

_head = pl.pallas_call(
    functools.partial(_layer_body, act="tanh"),
    out_shape=jax.ShapeDtypeStruct((N, D), jnp.float32),
)


def kernel(x, edge_index, batch, W1, b1, W2, b2, gamma, beta):
    src = edge_index[0].astype(jnp.int32)
    dst = edge_index[1].astype(jnp.int32)
    pad = EPAD - E
    src_p = jnp.concatenate([src, jnp.zeros((pad,), jnp.int32)])
    dst_p = jnp.concatenate([dst, jnp.full((pad,), N, jnp.int32)])
    zeros = jnp.zeros((NPAD, D), jnp.float32)

    b1r = b1.reshape(-1, 1, D)
    b2r = b2.reshape(-1, 1, D)
    gr = gamma.reshape(-1, 1, D)
    ber = beta.reshape(-1, 1, D)

    agg = _make_agg()
    h = x
    for i in range(3):
        p = agg(h, src_p, dst_p, zeros)
        z = _combine(h, p)
        h = _layer(z, W1[i], b1r[i], W2[i], b2r[i], gr[i], ber[i])
    p = agg(h, src_p, dst_p, zeros)
    z = _combine(h, p)
    return tuple(_head(z, W1[j], b1r[j], W2[j], b2r[j], gr[j], ber[j])
                 for j in range(3, 7))


# speedup vs baseline: 1.0297x; 1.0297x over previous
import functools

import jax
import jax.numpy as jnp
from jax import lax
from jax.experimental import pallas as pl
from jax.experimental.pallas import tpu as pltpu
from jax.experimental.pallas import tpu_sc as plsc

N = 10000
D = 128
E = 320000
NPAD = 10112
ROWS_PER_TILE = NPAD // 16
K = 128
NW = 32
GB = 2
CHUNKS = 80
EPW = CHUNKS * K
EPAD = NW * EPW
GROUPS = CHUNKS // GB


def _agg_body(h_hbm, src_hbm, dst_hbm, zeros_hbm, out_hbm,
              src_v, dst0, dst1, dst2, dst3, rows_v,
              sem_i, sem_g, sem_s, acc):
    c = lax.axis_index("c")
    s = lax.axis_index("s")
    wid = s * 2 + c
    dsts = (dst0, dst1, dst2, dst3)
    r0 = s * ROWS_PER_TILE
    pltpu.sync_copy(zeros_hbm.at[pl.ds(r0, ROWS_PER_TILE)],
                    acc.at[pl.ds(r0, ROWS_PER_TILE)])
    plsc.subcore_barrier()

    base = wid * EPW

    def fire_idx(i, s_slot, d_slot):
        pltpu.async_copy(src_hbm.at[pl.ds(base + i * K, K)],
                         src_v.at[s_slot], sem_i)
        pltpu.async_copy(dst_hbm.at[pl.ds(base + i * K, K)],
                         dsts[d_slot], sem_i)

    def wait_idx(s_slot):
        pltpu.make_async_copy(src_hbm.at[pl.ds(0, K)], src_v.at[s_slot],
                              sem_i).wait()
        pltpu.make_async_copy(src_hbm.at[pl.ds(0, K)], src_v.at[s_slot],
                              sem_i).wait()

    def fire_gather_b(s_slot):
        pltpu.async_copy(h_hbm.at[src_v.at[s_slot]], rows_v.at[s_slot],
                         sem_g)

    def wait_gather(r):
        pltpu.make_async_copy(zeros_hbm.at[pl.ds(0, K)], rows_v.at[r],
                              sem_g).wait()

    def fire_scatter(r, d):
        pltpu.async_copy(rows_v.at[r], acc.at[dsts[d]], sem_s, add=True)

    def wait_scatter(r):
        pltpu.make_async_copy(zeros_hbm.at[pl.ds(0, K)], rows_v.at[r],
                              sem_s).wait()


    pltpu.sync_copy(src_hbm.at[pl.ds(base, K)], src_v.at[0])
    pltpu.sync_copy(dst_hbm.at[pl.ds(base, K)], dsts[0])
    fire_gather_b(0)
    fire_idx(1, 1, 1)

    def chunk_step(i, b, first, last):
        r = b % 2
        rn = (b + 1) % 2
        wait_gather(r)
        fire_scatter(r, b % 4)
        if not last[1]:
            fire_idx(i + 2, r, (b + 2) % 4)
        if not first:
            wait_scatter(rn)
        if not last[0]:
            wait_idx(rn)
            fire_gather_b(rn)

    NL = (False, False)
    for b in range(4):
        chunk_step(b, b, b == 0, NL)

    def quad(j, carry):
        for b in range(4):
            chunk_step(j * 4 + b, b, False, NL)
        return carry

    lax.fori_loop(1, CHUNKS // 4 - 1, quad, 0)

    for b in range(4):
        i = CHUNKS - 4 + b
        chunk_step(i, b, False, (i + 1 >= CHUNKS, i + 2 >= CHUNKS))

    wait_scatter((CHUNKS - 1) % 2)
    plsc.subcore_barrier()
    pltpu.sync_copy(acc.at[pl.ds(r0, ROWS_PER_TILE)],
                    out_hbm.at[c, pl.ds(r0, ROWS_PER_TILE)])


@functools.cache
def _make_agg():
    return pl.kernel(
        _agg_body,
        out_type=jax.ShapeDtypeStruct((2, NPAD, D), jnp.float32),
        mesh=plsc.VectorSubcoreMesh(core_axis_name="c", subcore_axis_name="s"),
        scratch_types=[
            pltpu.VMEM((2, K), jnp.int32),
            pltpu.VMEM((K,), jnp.int32),
            pltpu.VMEM((K,), jnp.int32),
            pltpu.VMEM((K,), jnp.int32),
            pltpu.VMEM((K,), jnp.int32),
            pltpu.VMEM((2, K, D), jnp.float32),
            pltpu.SemaphoreType.DMA,
            pltpu.SemaphoreType.DMA,
            pltpu.SemaphoreType.DMA,
            pltpu.VMEM_SHARED((NPAD, D), jnp.float32),
        ],
    )


def _bn(z, g, b):
    mu = jnp.mean(z, axis=0, keepdims=True)
    var = jnp.mean((z - mu) ** 2, axis=0, keepdims=True)
    return (z - mu) * lax.rsqrt(var + 1e-5) * g + b


def _mlp(z, w1, b1, w2, b2):
    z = jnp.maximum(
        jnp.dot(z, w1, preferred_element_type=jnp.float32) + b1, 0.0)
    return jnp.dot(z, w2, preferred_element_type=jnp.float32) + b2


def _combine_body(h_ref, p_ref, z_ref):
    z_ref[...] = h_ref[...] + p_ref[0, :N, :] + p_ref[1, :N, :]


_combine = pl.pallas_call(
    _combine_body,
    out_shape=jax.ShapeDtypeStruct((N, D), jnp.float32),
)


def _layer_body(z_ref, w1_ref, b1_ref, w2_ref, b2_ref, g_ref, be_ref,
                out_ref, *, act):
    z = _mlp(z_ref[...], w1_ref[...], b1_ref[...], w2_ref[...], b2_ref[...])
    z = jnp.maximum(z, 0.0) if act == "relu" else jnp.tanh(z)
    out_ref[...] = _bn(z, g_ref[...], be_ref[...])


_layer = pl.pallas_call(
    functools.partial(_layer_body, act="relu"),
    out_shape=jax.ShapeDtypeStruct((N, D), jnp.float32),
)

_head = pl.pallas_call(
    functools.partial(_layer_body, act="tanh"),
    out_shape=jax.ShapeDtypeStruct((N, D), jnp.float32),
)


def kernel(x, edge_index, batch, W1, b1, W2, b2, gamma, beta):
    src = edge_index[0].astype(jnp.int32)
    dst = edge_index[1].astype(jnp.int32)
    pad = EPAD - E
    src_p = jnp.concatenate([src, jnp.zeros((pad,), jnp.int32)])
    dst_p = jnp.concatenate([dst, jnp.full((pad,), N, jnp.int32)])
    zeros = jnp.zeros((NPAD, D), jnp.float32)

    b1r = b1.reshape(-1, 1, D)
    b2r = b2.reshape(-1, 1, D)
    gr = gamma.reshape(-1, 1, D)
    ber = beta.reshape(-1, 1, D)

    agg = _make_agg()
    h = x
    for i in range(3):
        p = agg(h, src_p, dst_p, zeros)
        z = _combine(h, p)
        h = _layer(z, W1[i], b1r[i], W2[i], b2r[i], gr[i], ber[i])
    p = agg(h, src_p, dst_p, zeros)
    z = _combine(h, p)
    return tuple(_head(z, W1[j], b1r[j], W2[j], b2r[j], gr[j], ber[j])
                 for j in range(3, 7))
